# Initial kernel scaffold; baseline (speedup 1.0000x reference)
import jax
import jax.numpy as jnp
from jax.experimental import pallas as pl


def kernel(embeddings, codebook):
    raise NotImplementedError("write your pallas kernel here")


# speedup vs baseline: 1.4423x; 1.4423x over previous
import jax
import jax.numpy as jnp
from jax import lax
from jax.experimental import pallas as pl

_B, _E, _HW, _K = 16, 64, 1024, 1024


def _vq_body(x_ref, cb_ref, out_ref):
    x = x_ref[0].reshape(_E, _HW)
    cb = cb_ref[...]
    sq1 = jnp.sum(x * x, axis=0)[None, :]
    sq2 = jnp.sum(cb * cb, axis=1)[:, None]
    cross = lax.dot_general(cb, x, (((1,), (0,)), ((), ())),
                            preferred_element_type=jnp.float32)
    dists = sq1 - 2.0 * cross + sq2
    m = jnp.min(dists, axis=0, keepdims=True)
    iota = lax.broadcasted_iota(jnp.int32, (_K, _HW), 0)
    idx = jnp.min(jnp.where(dists == m, iota, _K), axis=0)
    onehot = (iota == idx[None, :]).astype(jnp.float32)
    dec = lax.dot_general(cb, onehot, (((0,), (0,)), ((), ())),
                          preferred_element_type=jnp.float32)
    out_ref[0] = dec.reshape(_E, 32, 32)


_vq_call = pl.pallas_call(
    _vq_body,
    grid=(_B,),
    in_specs=[
        pl.BlockSpec((1, _E, 32, 32), lambda b: (b, 0, 0, 0)),
        pl.BlockSpec((_K, _E), lambda b: (0, 0)),
    ],
    out_specs=pl.BlockSpec((1, _E, 32, 32), lambda b: (b, 0, 0, 0)),
    out_shape=jax.ShapeDtypeStruct((_B, _E, 32, 32), jnp.float32),
)


def kernel(embeddings, codebook):
    return _vq_call(embeddings, codebook)
